# baseline (speedup 1.0000x reference)
import jax
import jax.numpy as jnp
from jax import lax
from jax.experimental import pallas as pl
from jax.experimental.pallas import tpu as pltpu
from jax.experimental.pallas import tpu_sc as plsc

_NC, _NS, _L = 2, 16, 16
_NW = _NC * _NS
_BATCH = 16
_STREAMS = 8
_CHAN = 2
_T = 65536
_SEGS = _BATCH * _STREAMS * _CHAN
_SEG_PER_W = _SEGS // _NW
_NBUF = 7
_CH = 16384
_CPS = _T // _CH
_N = _SEG_PER_W * _CPS
_ZN = 4096
_ZPC = _CH // _ZN
_GLEAD = 0


def _body(audio_hbm, src_hbm, out_hbm, src_v, zbuf, shbuf, *rest):
    gsems = rest[:_NBUF]
    ssems = rest[_NBUF:2 * _NBUF]
    zsem = rest[2 * _NBUF]
    sid = lax.axis_index("s")
    wid = sid * _NC + lax.axis_index("c")
    bufs = [shbuf.at[pl.ds((sid * _NBUF + i) * _CH, _CH)] for i in range(_NBUF)]

    pltpu.sync_copy(src_hbm, src_v)

    flags = []
    coords = []
    for j in range(_SEG_PER_W):
        seg = wid * _SEG_PER_W + j
        b = seg // (_STREAMS * _CHAN)
        s = (seg // _CHAN) % _STREAMS
        c = seg % _CHAN
        coords.append((b, s, c))
        src_b = plsc.load_gather(src_v, [jnp.full((_L,), b, jnp.int32)])
        flags.append(src_b[0] == s)

    def pred(k):
        return flags[k // _CPS]

    def dst_slice(k, off, size):
        b, s, c = coords[k // _CPS]
        return out_hbm.at[b, s, c, pl.ds((k % _CPS) * _CH + off, size)]

    def gdesc(k):
        b, s, c = coords[k // _CPS]
        return (audio_hbm.at[b, s, c, pl.ds((k % _CPS) * _CH, _CH)],
                bufs[k % _NBUF], gsems[k % _NBUF])

    def sdesc(k):
        return (bufs[k % _NBUF], dst_slice(k, 0, _CH), ssems[k % _NBUF])

    for k in range(_GLEAD):

        @pl.when(jnp.logical_not(pred(k)))
        def _(k=k):
            pltpu.async_copy(*gdesc(k))

    zero = jnp.zeros((_L,), jnp.float32)

    def _zfill(i, carry):
        for u in range(16):
            zbuf[pl.ds((i * 16 + u) * _L, _L)] = zero
        return carry

    lax.fori_loop(0, _ZN // (_L * 16), _zfill, 0)

    for k in range(_N):
        q = k + _GLEAD
        if _GLEAD <= q < _N:
            if q - _NBUF >= 0:

                @pl.when(jnp.logical_not(pred(q - _NBUF)))
                def _(q=q):
                    pltpu.make_async_copy(*sdesc(q - _NBUF)).wait()

            @pl.when(jnp.logical_not(pred(q)))
            def _(q=q):
                pltpu.async_copy(*gdesc(q))

        @pl.when(pred(k))
        def _(k=k):
            for h in range(_ZPC):
                pltpu.async_copy(zbuf, dst_slice(k, h * _ZN, _ZN), zsem)

        @pl.when(jnp.logical_not(pred(k)))
        def _(k=k):
            pltpu.make_async_copy(*gdesc(k)).wait()
            pltpu.async_copy(*sdesc(k))

    for k in range(max(0, _N - _NBUF), _N):

        @pl.when(jnp.logical_not(pred(k)))
        def _(k=k):
            pltpu.make_async_copy(*sdesc(k)).wait()

    for k in range(_N):

        @pl.when(pred(k))
        def _(k=k):
            for h in range(_ZPC):
                pltpu.make_async_copy(zbuf, dst_slice(k, h * _ZN, _ZN),
                                      zsem).wait()


def kernel(audio, source_to_zero):
    mesh = plsc.VectorSubcoreMesh(core_axis_name="c", subcore_axis_name="s")
    out = pl.kernel(
        _body,
        out_type=jax.ShapeDtypeStruct(audio.shape, audio.dtype),
        mesh=mesh,
        scratch_types=[
            pltpu.VMEM((_BATCH,), jnp.int32),
            pltpu.VMEM((_ZN,), jnp.float32),
            pltpu.VMEM_SHARED((_NS * _NBUF * _CH,), jnp.float32),
        ] + [pltpu.SemaphoreType.DMA] * (2 * _NBUF + 1),
        compiler_params=pltpu.CompilerParams(needs_layout_passes=False),
    )(audio, source_to_zero)
    return out


# speedup vs baseline: 1.1223x; 1.1223x over previous
import jax
import jax.numpy as jnp
from jax import lax
from jax.experimental import pallas as pl
from jax.experimental.pallas import tpu as pltpu
from jax.experimental.pallas import tpu_sc as plsc

_NC, _NS, _L = 2, 16, 16
_NW = _NC * _NS
_BATCH = 16
_STREAMS = 8
_CHAN = 2
_T = 65536
_SEGS = _BATCH * _STREAMS * _CHAN
_SEG_PER_W = _SEGS // _NW
_NBUF = 5
_CH = 16384
_CPS = _T // _CH
_N = _SEG_PER_W * _CPS
_ZN = 4096
_ZPC = _CH // _ZN
_GLEAD = 1


def _body(audio_hbm, src_hbm, out_hbm, src_v, zbuf, shbuf, *rest):
    gsems = rest[:_NBUF]
    ssems = rest[_NBUF:2 * _NBUF]
    zsem = rest[2 * _NBUF]
    sid = lax.axis_index("s")
    wid = sid * _NC + lax.axis_index("c")
    bufs = [shbuf.at[pl.ds((sid * _NBUF + i) * _CH, _CH)] for i in range(_NBUF)]

    pltpu.sync_copy(src_hbm, src_v)

    flags = []
    coords = []
    for j in range(_SEG_PER_W):
        seg = wid * _SEG_PER_W + j
        b = seg // (_STREAMS * _CHAN)
        s = (seg // _CHAN) % _STREAMS
        c = seg % _CHAN
        coords.append((b, s, c))
        src_b = plsc.load_gather(src_v, [jnp.full((_L,), b, jnp.int32)])
        flags.append(src_b[0] == s)

    def pred(k):
        return flags[k // _CPS]

    def dst_slice(k, off, size):
        b, s, c = coords[k // _CPS]
        return out_hbm.at[b, s, c, pl.ds((k % _CPS) * _CH + off, size)]

    def gdesc(k):
        b, s, c = coords[k // _CPS]
        return (audio_hbm.at[b, s, c, pl.ds((k % _CPS) * _CH, _CH)],
                bufs[k % _NBUF], gsems[k % _NBUF])

    def sdesc(k):
        return (bufs[k % _NBUF], dst_slice(k, 0, _CH), ssems[k % _NBUF])

    for k in range(_GLEAD):

        @pl.when(jnp.logical_not(pred(k)))
        def _(k=k):
            pltpu.async_copy(*gdesc(k))

    zero = jnp.zeros((_L,), jnp.float32)

    def _zfill(i, carry):
        for u in range(16):
            zbuf[pl.ds((i * 16 + u) * _L, _L)] = zero
        return carry

    lax.fori_loop(0, _ZN // (_L * 16), _zfill, 0)

    for k in range(_N):
        q = k + _GLEAD
        if _GLEAD <= q < _N:
            if q - _NBUF >= 0:

                @pl.when(jnp.logical_not(pred(q - _NBUF)))
                def _(q=q):
                    pltpu.make_async_copy(*sdesc(q - _NBUF)).wait()

            @pl.when(jnp.logical_not(pred(q)))
            def _(q=q):
                pltpu.async_copy(*gdesc(q))

        @pl.when(pred(k))
        def _(k=k):
            for h in range(_ZPC):
                pltpu.async_copy(zbuf, dst_slice(k, h * _ZN, _ZN), zsem)

        @pl.when(jnp.logical_not(pred(k)))
        def _(k=k):
            pltpu.make_async_copy(*gdesc(k)).wait()
            pltpu.async_copy(*sdesc(k))

    for k in range(max(0, _N - _NBUF), _N):

        @pl.when(jnp.logical_not(pred(k)))
        def _(k=k):
            pltpu.make_async_copy(*sdesc(k)).wait()

    for k in range(_N):

        @pl.when(pred(k))
        def _(k=k):
            for h in range(_ZPC):
                pltpu.make_async_copy(zbuf, dst_slice(k, h * _ZN, _ZN),
                                      zsem).wait()


def kernel(audio, source_to_zero):
    mesh = plsc.VectorSubcoreMesh(core_axis_name="c", subcore_axis_name="s")
    out = pl.kernel(
        _body,
        out_type=jax.ShapeDtypeStruct(audio.shape, audio.dtype),
        mesh=mesh,
        scratch_types=[
            pltpu.VMEM((_BATCH,), jnp.int32),
            pltpu.VMEM((_ZN,), jnp.float32),
            pltpu.VMEM_SHARED((_NS * _NBUF * _CH,), jnp.float32),
        ] + [pltpu.SemaphoreType.DMA] * (2 * _NBUF + 1),
        compiler_params=pltpu.CompilerParams(needs_layout_passes=False),
    )(audio, source_to_zero)
    return out
